# Initial kernel scaffold; baseline (speedup 1.0000x reference)
import jax
import jax.numpy as jnp
from jax.experimental import pallas as pl


def kernel(tokens, table):
    raise NotImplementedError("write your pallas kernel here")


# speedup vs baseline: 1.2572x; 1.2572x over previous
import functools
import math

import jax
import jax.numpy as jnp
from jax import lax
from jax.experimental import pallas as pl
from jax.experimental.pallas import tpu as pltpu
from jax.experimental.pallas import tpu_sc as plsc

EMB = 32
SCALE = math.sqrt(float(EMB))

NC = 2
NS = 16
NW = NC * NS

B = 4096 * 200
BPW = B // NW
G = 128
NG = BPW // G

_mesh = plsc.VectorSubcoreMesh(core_axis_name="c", subcore_axis_name="s")


@functools.partial(
    pl.kernel,
    mesh=_mesh,
    compiler_params=pltpu.CompilerParams(use_tc_tiling_on_sc=False),
    out_type=jax.ShapeDtypeStruct((B, EMB), jnp.float32),
    scratch_types=[
        pltpu.VMEM((NG, G), jnp.int32),
        pltpu.VMEM((G, EMB), jnp.float32),
        pltpu.SemaphoreType.DMA,
    ],
)
def _embed(tok_hbm, table_hbm, out_hbm, idx_v, rows_v, sem):
    wid = lax.axis_index("s") * NC + lax.axis_index("c")
    base = wid * BPW

    pltpu.sync_copy(tok_hbm.at[wid], idx_v)

    def gather_body(g, carry):
        pltpu.async_copy(table_hbm.at[idx_v.at[g]], rows_v, sem).wait()

        def scale_row(r, c):
            for h in range(2):
                sl = (r, pl.ds(h * 16, 16))
                rows_v[sl] = rows_v[sl] * SCALE
            return c

        lax.fori_loop(0, G, scale_row, 0, unroll=4)
        pltpu.sync_copy(rows_v, out_hbm.at[pl.ds(base + g * G, G)])
        return carry

    lax.fori_loop(0, NG, gather_body, 0)


def kernel(tokens, table):
    flat = tokens.reshape(NW, NG, G).astype(jnp.int32)
    out = _embed(flat, table)
    return out.reshape(tokens.shape + (EMB,))


# speedup vs baseline: 1.4810x; 1.1780x over previous
import functools
import math

import jax
import jax.numpy as jnp
from jax import lax
from jax.experimental import pallas as pl
from jax.experimental.pallas import tpu as pltpu
from jax.experimental.pallas import tpu_sc as plsc

EMB = 32
SCALE = math.sqrt(float(EMB))

NC = 2
NS = 16
NW = NC * NS

B = 4096 * 200
BPW = B // NW
G = 128
NG = BPW // G
K = 5
GR = K * G
NGRP = NG // K
NBUF = 4

_mesh = plsc.VectorSubcoreMesh(core_axis_name="c", subcore_axis_name="s")


@functools.partial(
    pl.kernel,
    mesh=_mesh,
    compiler_params=pltpu.CompilerParams(use_tc_tiling_on_sc=False),
    out_type=jax.ShapeDtypeStruct((B, EMB), jnp.float32),
    scratch_types=[
        pltpu.VMEM((NG, G), jnp.int32),
        pltpu.VMEM((GR, EMB), jnp.float32),
        pltpu.VMEM((GR, EMB), jnp.float32),
        pltpu.VMEM((GR, EMB), jnp.float32),
        pltpu.VMEM((GR, EMB), jnp.float32),
        pltpu.SemaphoreType.DMA,
        pltpu.SemaphoreType.DMA,
        pltpu.SemaphoreType.DMA,
        pltpu.SemaphoreType.DMA,
        pltpu.SemaphoreType.DMA,
        pltpu.SemaphoreType.DMA,
        pltpu.SemaphoreType.DMA,
        pltpu.SemaphoreType.DMA,
    ],
)
def _embed(tok_hbm, table_hbm, out_hbm, idx_v,
           rb0, rb1, rb2, rb3, sg0, sg1, sg2, sg3, so0, so1, so2, so3):
    rows = [rb0, rb1, rb2, rb3]
    sg = [sg0, sg1, sg2, sg3]
    so = [so0, so1, so2, so3]
    wid = lax.axis_index("s") * NC + lax.axis_index("c")
    base = wid * BPW

    pltpu.sync_copy(tok_hbm.at[wid], idx_v)

    def fire(g, b):
        for j in range(K):
            pltpu.make_async_copy(
                table_hbm.at[idx_v.at[g * K + j]],
                rows[b].at[pl.ds(j * G, G)],
                sg[b],
            ).start()

    def drain_gathers(b):
        pltpu.make_async_copy(out_hbm.at[pl.ds(0, GR)], rows[b], sg[b]).wait()

    def wait_out(b):
        pltpu.make_async_copy(out_hbm.at[pl.ds(0, GR)], rows[b], so[b]).wait()

    for b in range(NBUF - 1):
        fire(b, b)

    def outer(t, carry):
        for b in range(NBUF):
            g = t * NBUF + b
            drain_gathers(b)

            def scale_row(r, c):
                for h in range(2):
                    sl = (r, pl.ds(h * 16, 16))
                    rows[b][sl] = rows[b][sl] * SCALE
                return c

            lax.fori_loop(0, GR, scale_row, 0, unroll=8)

            pltpu.make_async_copy(
                rows[b], out_hbm.at[pl.ds(base + g * GR, GR)], so[b]
            ).start()

            bp = (b - 1) % NBUF

            @pl.when(g >= 1)
            def _():
                wait_out(bp)

            @pl.when(g + NBUF - 1 < NGRP)
            def _():
                fire(g + NBUF - 1, bp)
        return carry

    lax.fori_loop(0, NGRP // NBUF, outer, 0)
    wait_out(NBUF - 1)


def kernel(tokens, table):
    flat = tokens.reshape(NW, NG, G).astype(jnp.int32)
    out = _embed(flat, table)
    return out.reshape(tokens.shape + (EMB,))


# speedup vs baseline: 2.0258x; 1.3679x over previous
import functools
import math

import jax
import jax.numpy as jnp
from jax import lax
from jax.experimental import pallas as pl
from jax.experimental.pallas import tpu as pltpu
from jax.experimental.pallas import tpu_sc as plsc

EMB = 32
SCALE = math.sqrt(float(EMB))

NC = 2
NS = 16
NW = NC * NS

B = 4096 * 200
BPW = B // NW
G = 128
NG = BPW // G
K = 5
GR = K * G
NGRP = NG // K
NBUF = 4

_mesh = plsc.VectorSubcoreMesh(core_axis_name="c", subcore_axis_name="s")


@functools.partial(
    pl.kernel,
    mesh=_mesh,
    compiler_params=pltpu.CompilerParams(use_tc_tiling_on_sc=False),
    out_type=jax.ShapeDtypeStruct((B, 128), jnp.float32),
    scratch_types=[
        pltpu.VMEM((NG, G), jnp.int32),
        pltpu.VMEM((GR, EMB), jnp.float32),
        pltpu.VMEM((GR, EMB), jnp.float32),
        pltpu.VMEM((GR, EMB), jnp.float32),
        pltpu.VMEM((GR, EMB), jnp.float32),
        pltpu.SemaphoreType.DMA,
        pltpu.SemaphoreType.DMA,
        pltpu.SemaphoreType.DMA,
        pltpu.SemaphoreType.DMA,
        pltpu.SemaphoreType.DMA,
        pltpu.SemaphoreType.DMA,
        pltpu.SemaphoreType.DMA,
        pltpu.SemaphoreType.DMA,
    ],
)
def _embed(tok_hbm, table_hbm, out_hbm, idx_v,
           rb0, rb1, rb2, rb3, sg0, sg1, sg2, sg3, so0, so1, so2, so3):
    rows = [rb0, rb1, rb2, rb3]
    sg = [sg0, sg1, sg2, sg3]
    so = [so0, so1, so2, so3]
    wid = lax.axis_index("s") * NC + lax.axis_index("c")
    base = wid * BPW

    pltpu.sync_copy(tok_hbm.at[wid], idx_v)

    def fire(g, b):
        for j in range(K):
            pltpu.make_async_copy(
                table_hbm.at[idx_v.at[g * K + j]],
                rows[b].at[pl.ds(j * G, G)],
                sg[b],
            ).start()

    def drain_gathers(b):
        pltpu.make_async_copy(out_hbm.at[pl.ds(0, GR), pl.ds(0, EMB)],
                              rows[b], sg[b]).wait()

    def wait_out(b):
        pltpu.make_async_copy(out_hbm.at[pl.ds(0, GR), pl.ds(0, EMB)],
                              rows[b], so[b]).wait()

    for b in range(NBUF - 1):
        fire(b, b)

    def outer(t, carry):
        for b in range(NBUF):
            g = t * NBUF + b
            drain_gathers(b)

            def scale_row(r, c):
                for h in range(2):
                    sl = (r, pl.ds(h * 16, 16))
                    rows[b][sl] = rows[b][sl] * SCALE
                return c

            lax.fori_loop(0, GR, scale_row, 0, unroll=8)

            pltpu.make_async_copy(
                rows[b],
                out_hbm.at[pl.ds(base + g * GR, GR), pl.ds(0, EMB)],
                so[b],
            ).start()

            bp = (b - 1) % NBUF

            @pl.when(g >= 1)
            def _():
                wait_out(bp)

            @pl.when(g + NBUF - 1 < NGRP)
            def _():
                fire(g + NBUF - 1, bp)
        return carry

    lax.fori_loop(0, NGRP // NBUF, outer, 0)
    wait_out(NBUF - 1)


def kernel(tokens, table):
    flat = tokens.reshape(NW, NG, G).astype(jnp.int32)
    out = _embed(flat, table)
    return out[:, :EMB].reshape(tokens.shape + (EMB,))
